# baseline (speedup 1.0000x reference)
import functools

import jax
import jax.numpy as jnp
from jax import lax
from jax.experimental import pallas as pl
from jax.experimental.pallas import tpu as pltpu
from jax.experimental.pallas import tpu_sc as plsc

_B, _C, _N, _K = 16, 768, 2048, 128
_R = _B * _C
_NW = 32
_ROWS_PER_W = _R // _NW
_VPC = _K // 16
_IL = 2


def _sc_sort16(v, descending):
  if descending:
    return -lax.sort(-v, dimension=0)
  return lax.sort(v, dimension=0)


def _bitonic_clean(vs, sortfn, descending):
  vs = list(vs)
  m = len(vs)
  s = m // 2
  while s >= 1:
    nxt = list(vs)
    for blk in range(0, m, 2 * s):
      for i in range(blk, blk + s):
        a, b = vs[i], vs[i + s]
        if descending:
          nxt[i] = jnp.maximum(a, b)
          nxt[i + s] = jnp.minimum(a, b)
        else:
          nxt[i] = jnp.minimum(a, b)
          nxt[i + s] = jnp.maximum(a, b)
    vs = nxt
    s //= 2
  return [sortfn(v, descending) for v in vs]


def _merge_keep_all(run_a, run_b, sortfn, descending):
  m = len(run_a)
  if descending:
    hi = [jnp.maximum(run_a[i], run_b[i]) for i in range(m)]
    lo = [jnp.minimum(run_a[i], run_b[i]) for i in range(m)]
  else:
    hi = [jnp.minimum(run_a[i], run_b[i]) for i in range(m)]
    lo = [jnp.maximum(run_a[i], run_b[i]) for i in range(m)]
  return (_bitonic_clean(hi, sortfn, descending)
          + _bitonic_clean(lo, sortfn, descending))


def _merge_keep_top(run_a, run_b, sortfn):
  m = len(run_a)
  hi = [jnp.maximum(run_a[i], run_b[i]) for i in range(m)]
  return _bitonic_clean(hi, sortfn, True)


def _build_sorted_run(vs, sortfn, descending):
  if len(vs) == 1:
    return [sortfn(vs[0], descending)]
  h = len(vs) // 2
  run_a = _build_sorted_run(vs[:h], sortfn, descending)
  run_b = _build_sorted_run(vs[h:], sortfn, not descending)
  return _merge_keep_all(run_a, run_b, sortfn, descending)


@functools.lru_cache(maxsize=1)
def _make_sc_topk():
  mesh = plsc.VectorSubcoreMesh(
      core_axis_name="c", subcore_axis_name="s", num_cores=2, num_subcores=16)

  @functools.partial(
      pl.kernel,
      out_type=jax.ShapeDtypeStruct((_R, _K), jnp.float32),
      mesh=mesh,
      scratch_types=[
          pltpu.VMEM((_IL, _N), jnp.float32),
          pltpu.VMEM((_IL, _N), jnp.float32),
          pltpu.VMEM((_IL, _K), jnp.float32),
          pltpu.VMEM((_IL, _K), jnp.float32),
          pltpu.SemaphoreType.DMA,
          pltpu.SemaphoreType.DMA,
          pltpu.SemaphoreType.DMA,
          pltpu.SemaphoreType.DMA,
      ],
      compiler_params=pltpu.CompilerParams(needs_layout_passes=False),
  )
  def topk_rows(x_hbm, out_hbm, in_a, in_b, o0, o1,
                sem_a, sem_b, sem_o0, sem_o1):
    wid = lax.axis_index("s") * 2 + lax.axis_index("c")
    base = wid * _ROWS_PER_W

    def compute_il(buf, out_buf):
      runs = []
      for k in range(_IL):
        first = [buf[k, pl.ds(16 * i, 16)] for i in range(_VPC)]
        runs.append(tuple(_build_sorted_run(first, _sc_sort16, True)))

      def chunk_body(ci, carry):
        outs = []
        for k in range(_IL):
          rk = list(carry[k * _VPC:(k + 1) * _VPC])
          vs = [buf[k, pl.ds(ci * _K + 16 * i, 16)] for i in range(_VPC)]
          nr = _build_sorted_run(vs, _sc_sort16, False)
          outs.extend(_merge_keep_top(rk, nr, _sc_sort16))
        return tuple(outs)

      carry = lax.fori_loop(1, _N // _K, chunk_body, sum(runs, ()))
      for k in range(_IL):
        for i in range(_VPC):
          out_buf[k, pl.ds(16 * i, 16)] = carry[k * _VPC + i]

    pltpu.sync_copy(x_hbm.at[pl.ds(base, _IL)], in_a)

    def body(q, carry):
      r0 = base + 2 * _IL * q
      dma_b = pltpu.async_copy(x_hbm.at[pl.ds(r0 + _IL, _IL)], in_b, sem_b)

      @pl.when(q > 0)
      def _wait_o0():
        pltpu.make_async_copy(o0, out_hbm.at[pl.ds(r0, _IL)], sem_o0).wait()

      compute_il(in_a, o0)
      pltpu.async_copy(o0, out_hbm.at[pl.ds(r0, _IL)], sem_o0)
      dma_b.wait()

      nxt = jnp.minimum(r0 + 2 * _IL, _R - _IL)
      dma_a = pltpu.async_copy(x_hbm.at[pl.ds(nxt, _IL)], in_a, sem_a)

      @pl.when(q > 0)
      def _wait_o1():
        pltpu.make_async_copy(o1, out_hbm.at[pl.ds(r0, _IL)], sem_o1).wait()

      compute_il(in_b, o1)
      pltpu.async_copy(o1, out_hbm.at[pl.ds(r0 + _IL, _IL)], sem_o1)
      dma_a.wait()
      return carry

    lax.fori_loop(0, _ROWS_PER_W // (2 * _IL), body, 0)
    pltpu.make_async_copy(o0, out_hbm.at[pl.ds(base, _IL)], sem_o0).wait()
    pltpu.make_async_copy(o1, out_hbm.at[pl.ds(base, _IL)], sem_o1).wait()

  return topk_rows


def kernel(x):
  rows = x.reshape(_R, _N)
  out = _make_sc_topk()(rows)
  return out.reshape(_B, _C, _K)


# speedup vs baseline: 3.1338x; 3.1338x over previous
import functools

import jax
import jax.numpy as jnp
from jax import lax
from jax.experimental import pallas as pl
from jax.experimental.pallas import tpu as pltpu
from jax.experimental.pallas import tpu_sc as plsc

_B, _C, _N, _K = 16, 768, 2048, 128
_R = _B * _C
_NW = 32
_ROWS_PER_W = _R // _NW
_VPC = _K // 16
_IL = 2


def _sc_sort16(v, descending):
  return plsc.sort_key_val(v, v, descending=descending)[0]


def _bitonic_clean(vs, sortfn, descending):
  vs = list(vs)
  m = len(vs)
  s = m // 2
  while s >= 1:
    nxt = list(vs)
    for blk in range(0, m, 2 * s):
      for i in range(blk, blk + s):
        a, b = vs[i], vs[i + s]
        if descending:
          nxt[i] = jnp.maximum(a, b)
          nxt[i + s] = jnp.minimum(a, b)
        else:
          nxt[i] = jnp.minimum(a, b)
          nxt[i + s] = jnp.maximum(a, b)
    vs = nxt
    s //= 2
  return [sortfn(v, descending) for v in vs]


def _merge_keep_all(run_a, run_b, sortfn, descending):
  m = len(run_a)
  if descending:
    hi = [jnp.maximum(run_a[i], run_b[i]) for i in range(m)]
    lo = [jnp.minimum(run_a[i], run_b[i]) for i in range(m)]
  else:
    hi = [jnp.minimum(run_a[i], run_b[i]) for i in range(m)]
    lo = [jnp.maximum(run_a[i], run_b[i]) for i in range(m)]
  return (_bitonic_clean(hi, sortfn, descending)
          + _bitonic_clean(lo, sortfn, descending))


def _merge_keep_top(run_a, run_b, sortfn):
  m = len(run_a)
  hi = [jnp.maximum(run_a[i], run_b[i]) for i in range(m)]
  return _bitonic_clean(hi, sortfn, True)


def _build_sorted_run(vs, sortfn, descending):
  if len(vs) == 1:
    return [sortfn(vs[0], descending)]
  h = len(vs) // 2
  run_a = _build_sorted_run(vs[:h], sortfn, descending)
  run_b = _build_sorted_run(vs[h:], sortfn, not descending)
  return _merge_keep_all(run_a, run_b, sortfn, descending)


@functools.lru_cache(maxsize=1)
def _make_sc_topk():
  mesh = plsc.VectorSubcoreMesh(
      core_axis_name="c", subcore_axis_name="s", num_cores=2, num_subcores=16)

  @functools.partial(
      pl.kernel,
      out_type=jax.ShapeDtypeStruct((_R, _K), jnp.float32),
      mesh=mesh,
      scratch_types=[
          pltpu.VMEM((_IL, _N), jnp.float32),
          pltpu.VMEM((_IL, _N), jnp.float32),
          pltpu.VMEM((_IL, _K), jnp.float32),
          pltpu.VMEM((_IL, _K), jnp.float32),
          pltpu.SemaphoreType.DMA,
          pltpu.SemaphoreType.DMA,
          pltpu.SemaphoreType.DMA,
          pltpu.SemaphoreType.DMA,
      ],
      compiler_params=pltpu.CompilerParams(needs_layout_passes=False),
  )
  def topk_rows(x_hbm, out_hbm, in_a, in_b, o0, o1,
                sem_a, sem_b, sem_o0, sem_o1):
    wid = lax.axis_index("s") * 2 + lax.axis_index("c")
    base = wid * _ROWS_PER_W

    def compute_il(buf, out_buf):
      runs = []
      for k in range(_IL):
        first = [buf[k, pl.ds(16 * i, 16)] for i in range(_VPC)]
        runs.append(tuple(_build_sorted_run(first, _sc_sort16, True)))

      def chunk_body(ci, carry):
        outs = []
        for k in range(_IL):
          rk = list(carry[k * _VPC:(k + 1) * _VPC])
          vs = [buf[k, pl.ds(ci * _K + 16 * i, 16)] for i in range(_VPC)]
          nr = _build_sorted_run(vs, _sc_sort16, False)
          outs.extend(_merge_keep_top(rk, nr, _sc_sort16))
        return tuple(outs)

      carry = lax.fori_loop(1, _N // _K, chunk_body, sum(runs, ()))
      for k in range(_IL):
        for i in range(_VPC):
          out_buf[k, pl.ds(16 * i, 16)] = carry[k * _VPC + i]

    pltpu.sync_copy(x_hbm.at[pl.ds(base, _IL)], in_a)

    def body(q, carry):
      r0 = base + 2 * _IL * q
      dma_b = pltpu.async_copy(x_hbm.at[pl.ds(r0 + _IL, _IL)], in_b, sem_b)

      @pl.when(q > 0)
      def _wait_o0():
        pltpu.make_async_copy(o0, out_hbm.at[pl.ds(r0, _IL)], sem_o0).wait()

      compute_il(in_a, o0)
      pltpu.async_copy(o0, out_hbm.at[pl.ds(r0, _IL)], sem_o0)
      dma_b.wait()

      nxt = jnp.minimum(r0 + 2 * _IL, _R - _IL)
      dma_a = pltpu.async_copy(x_hbm.at[pl.ds(nxt, _IL)], in_a, sem_a)

      @pl.when(q > 0)
      def _wait_o1():
        pltpu.make_async_copy(o1, out_hbm.at[pl.ds(r0, _IL)], sem_o1).wait()

      compute_il(in_b, o1)
      pltpu.async_copy(o1, out_hbm.at[pl.ds(r0 + _IL, _IL)], sem_o1)
      dma_a.wait()
      return carry

    lax.fori_loop(0, _ROWS_PER_W // (2 * _IL), body, 0)
    pltpu.make_async_copy(o0, out_hbm.at[pl.ds(base, _IL)], sem_o0).wait()
    pltpu.make_async_copy(o1, out_hbm.at[pl.ds(base, _IL)], sem_o1).wait()

  return topk_rows


def kernel(x):
  rows = x.reshape(_R, _N)
  out = _make_sc_topk()(rows)
  return out.reshape(_B, _C, _K)
